# Initial kernel scaffold; baseline (speedup 1.0000x reference)
import jax
import jax.numpy as jnp
from jax.experimental import pallas as pl


def kernel(feat, edge_index, edge_weight, W0, a0, W1, a1):
    raise NotImplementedError("write your pallas kernel here")


# speedup vs baseline: 5.9037x; 5.9037x over previous
import functools

import jax
import jax.numpy as jnp
from jax import lax
from jax.experimental import pallas as pl
from jax.experimental.pallas import tpu as pltpu
from jax.experimental.pallas import tpu_sc as plsc

NC = 2
NS = 16
NW = NC * NS
K = 128
L = 16


def _mesh():
  return plsc.VectorSubcoreMesh(core_axis_name="c", subcore_axis_name="s")


def _degree_kernel(ch, full_ch, tail, np_pad):
  rows = np_pad // K

  def body(src_hbm, dst_hbm, zeros_hbm, degp_hbm, src_v, dst_v, dego_v,
           degi_v, ib, acc_o, acc_i):
    cid = lax.axis_index("c")
    sid = lax.axis_index("s")
    wid = cid * NS + sid
    pltpu.sync_copy(src_hbm.at[wid], src_v)
    pltpu.sync_copy(dst_hbm.at[wid], dst_v)
    pltpu.sync_copy(zeros_hbm, dego_v)
    pltpu.sync_copy(zeros_hbm, degi_v)

    @pl.when(sid == 0)
    def _():
      pltpu.sync_copy(zeros_hbm, acc_o)
      pltpu.sync_copy(zeros_hbm, acc_i)

    for t in range(rows // L):
      ib[pl.ds(t * L, L)] = lax.iota(jnp.int32, L) + t * L
    plsc.subcore_barrier()

    one16 = jnp.ones((L,), jnp.float32)

    def scat(j, t):
      sl = pl.ds(t * L, L)
      s16 = src_v[j, sl]
      d16 = dst_v[j, sl]
      plsc.addupdate_scatter(dego_v, [s16 >> 7, s16 & 127], one16)
      plsc.addupdate_scatter(degi_v, [d16 >> 7, d16 & 127], one16)

    def step(j, carry):
      for t in range(K // L):
        scat(j, t)
      return carry

    lax.fori_loop(0, full_ch, step, 0)
    for t in range(tail // L):
      scat(full_ch, t)

    pltpu.sync_copy(dego_v, acc_o.at[ib], add=True)
    pltpu.sync_copy(degi_v, acc_i.at[ib], add=True)
    plsc.subcore_barrier()

    @pl.when(sid == 0)
    def _():
      pltpu.sync_copy(acc_o, degp_hbm.at[cid, 0])
      pltpu.sync_copy(acc_i, degp_hbm.at[cid, 1])

  return pl.kernel(
      body,
      out_type=jax.ShapeDtypeStruct((NC, 2, rows, K), jnp.float32),
      mesh=_mesh(),
      compiler_params=pltpu.CompilerParams(needs_layout_passes=False),
      scratch_types=[
          pltpu.VMEM((ch, K), jnp.int32),
          pltpu.VMEM((ch, K), jnp.int32),
          pltpu.VMEM((rows, K), jnp.float32),
          pltpu.VMEM((rows, K), jnp.float32),
          pltpu.VMEM((rows,), jnp.int32),
          pltpu.VMEM_SHARED((rows, K), jnp.float32),
          pltpu.VMEM_SHARED((rows, K), jnp.float32),
      ],
  )


def _norm_body(degp_ref, norm_ref):
  d = degp_ref[...]
  deg = d[0] + d[1]
  norm_ref[...] = lax.rsqrt(jnp.clip(deg, 1.0, None))


def _coef_kernel(ch, np_pad):
  rows = np_pad // K

  def body(src_hbm, dst_hbm, ew_hbm, norm_hbm, cp_hbm, src_v, dst_v, ew_v,
           c_v, ns_v, nd_v):
    cid = lax.axis_index("c")
    sid = lax.axis_index("s")
    wid = cid * NS + sid
    pltpu.sync_copy(src_hbm.at[wid], src_v)
    pltpu.sync_copy(dst_hbm.at[wid], dst_v)
    pltpu.sync_copy(ew_hbm.at[wid], ew_v)
    pltpu.sync_copy(norm_hbm.at[0], ns_v)
    pltpu.sync_copy(norm_hbm.at[1], nd_v)

    def step(j, carry):
      for t in range(K // L):
        sl = pl.ds(t * L, L)
        s16 = src_v[j, sl]
        d16 = dst_v[j, sl]
        w16 = ew_v[j, sl]
        ns = plsc.load_gather(ns_v, [s16 >> 7, s16 & 127])
        nd = plsc.load_gather(nd_v, [d16 >> 7, d16 & 127])
        c_v[j, sl] = w16 * ns * nd
      return carry

    lax.fori_loop(0, ch, step, 0)
    pltpu.sync_copy(c_v, cp_hbm.at[wid])

  return pl.kernel(
      body,
      out_type=jax.ShapeDtypeStruct((NW, ch, K), jnp.float32),
      mesh=_mesh(),
      compiler_params=pltpu.CompilerParams(needs_layout_passes=False),
      scratch_types=[
          pltpu.VMEM((ch, K), jnp.int32),
          pltpu.VMEM((ch, K), jnp.int32),
          pltpu.VMEM((ch, K), jnp.float32),
          pltpu.VMEM((ch, K), jnp.float32),
          pltpu.VMEM((rows, K), jnp.float32),
          pltpu.VMEM((rows, K), jnp.float32),
      ],
  )


def _layer_kernel(n, np_pad, d, ch):
  nb = np_pad // NS
  nb_last = n - nb * (NS - 1)

  def body(h_hbm, src_hbm, dst_hbm, cp_hbm, zeros_hbm, part_hbm, src_v,
           srcb0, srcb1, dstb0, dstb1, cb0, cb1, rows0, rows1, acc, sem0,
           sem1):
    cid = lax.axis_index("c")
    sid = lax.axis_index("s")
    wid = cid * NS + sid
    pltpu.sync_copy(src_hbm.at[wid], src_v)

    @pl.when(sid < NS - 1)
    def _():
      pltpu.sync_copy(zeros_hbm, acc.at[pl.ds(nb * sid, nb)])

    @pl.when(sid == NS - 1)
    def _():
      pltpu.sync_copy(zeros_hbm.at[pl.ds(0, nb_last)],
                      acc.at[pl.ds(nb * (NS - 1), nb_last)])

    plsc.subcore_barrier()

    def issue(j, srcb, rows_ref, dstb, cb, sem):
      for t in range(K // L):
        sl = pl.ds(t * L, L)
        srcb[sl] = src_v[j, sl]
      pltpu.async_copy(h_hbm.at[srcb], rows_ref, sem)
      pltpu.async_copy(dst_hbm.at[wid, j], dstb, sem)
      pltpu.async_copy(cp_hbm.at[wid, j], cb, sem)

    def drain(j, srcb, rows_ref, dstb, cb, sem):
      pltpu.make_async_copy(h_hbm.at[srcb], rows_ref, sem).wait()
      pltpu.make_async_copy(dst_hbm.at[wid, j], dstb, sem).wait()
      pltpu.make_async_copy(cp_hbm.at[wid, j], cb, sem).wait()

    def scale_scatter(rows_ref, dstb, cb):
      def sk(k, carry):
        c16 = plsc.load_gather(cb, [jnp.full((L,), 0, jnp.int32) + k])
        for f in range(d // L):
          sl = pl.ds(f * L, L)
          rows_ref[k, sl] = rows_ref[k, sl] * c16
        return carry

      lax.fori_loop(0, K, sk, 0)
      pltpu.sync_copy(rows_ref, acc.at[dstb], add=True)

    def step1(j, carry):
      issue(j, srcb0, rows0, dstb0, cb0, sem0)
      drain(j, srcb0, rows0, dstb0, cb0, sem0)
      scale_scatter(rows0, dstb0, cb0)
      return carry

    lax.fori_loop(0, ch, step1, 0)
    plsc.subcore_barrier()

    @pl.when(sid < NS - 1)
    def _():
      sl = pl.ds(nb * sid, nb)
      pltpu.sync_copy(acc.at[sl], part_hbm.at[cid, sl])

    @pl.when(sid == NS - 1)
    def _():
      sl = pl.ds(nb * (NS - 1), nb_last)
      pltpu.sync_copy(acc.at[sl], part_hbm.at[cid, sl])

  return pl.kernel(
      body,
      out_type=jax.ShapeDtypeStruct((NC, n, d), jnp.float32),
      mesh=_mesh(),
      compiler_params=pltpu.CompilerParams(needs_layout_passes=False),
      scratch_types=[
          pltpu.VMEM((ch, K), jnp.int32),
          pltpu.VMEM((K,), jnp.int32),
          pltpu.VMEM((K,), jnp.int32),
          pltpu.VMEM((K,), jnp.int32),
          pltpu.VMEM((K,), jnp.int32),
          pltpu.VMEM((K,), jnp.float32),
          pltpu.VMEM((K,), jnp.float32),
          pltpu.VMEM((K, d), jnp.float32),
          pltpu.VMEM((K, d), jnp.float32),
          pltpu.VMEM_SHARED((n, d), jnp.float32),
          pltpu.SemaphoreType.DMA,
          pltpu.SemaphoreType.DMA,
      ],
  )


def _matmul_body(p_ref, w_ref, a_ref, h_ref, pool_ref):
  i = pl.program_id(0)
  agg = p_ref[0] + p_ref[1]
  out = jnp.dot(agg, w_ref[...], preferred_element_type=jnp.float32)
  out = jnp.where(out > 0, out, a_ref[...] * out)
  h_ref[...] = out

  @pl.when(i == 0)
  def _():
    pool_ref[...] = jnp.zeros_like(pool_ref)

  pool_ref[...] += jnp.sum(out, axis=0, keepdims=True)


def _matmul_call(part, w, a, n, d, bn):
  grid = n // bn
  return pl.pallas_call(
      _matmul_body,
      grid=(grid,),
      in_specs=[
          pl.BlockSpec((2, bn, d), lambda i: (0, i, 0)),
          pl.BlockSpec((d, d), lambda i: (0, 0)),
          pl.BlockSpec((1, d), lambda i: (0, 0)),
      ],
      out_specs=[
          pl.BlockSpec((bn, d), lambda i: (i, 0)),
          pl.BlockSpec((1, d), lambda i: (0, 0)),
      ],
      out_shape=[
          jax.ShapeDtypeStruct((n, d), jnp.float32),
          jax.ShapeDtypeStruct((1, d), jnp.float32),
      ],
  )(part, w, jnp.broadcast_to(a.reshape(1, 1), (1, d)))


def kernel(feat, edge_index, edge_weight, W0, a0, W1, a1):
  n, d = feat.shape
  e = edge_index.shape[1]
  per_w = -(-e // NW)
  ep_worker = (-(-per_w // K) + 1) * K
  ch = ep_worker // K
  full_ch, tail = divmod(per_w, K)
  np_pad = -(-n // (L * NS)) * (L * NS)

  feat = feat.astype(jnp.float32)
  src = edge_index[0].astype(jnp.int32)
  dst = edge_index[1].astype(jnp.int32)
  ew = edge_weight.astype(jnp.float32)

  pad_w = ep_worker - per_w
  spread = jnp.broadcast_to(jnp.arange(pad_w, dtype=jnp.int32) % n,
                            (NW, pad_w))

  def _shard(x, pad_idx):
    x = jnp.pad(x, (0, NW * per_w - e)).reshape(NW, per_w)
    fill = spread if pad_idx else jnp.zeros((NW, pad_w), x.dtype)
    return jnp.concatenate([x, fill.astype(x.dtype)],
                           axis=1).reshape(NW, ch, K)

  src_p = _shard(src, True)
  dst_p = _shard(dst, True)
  ew_p = _shard(ew, False)

  zeros_deg = jnp.zeros((np_pad // K, K), jnp.float32)
  zeros_acc = jnp.zeros((np_pad // NS, d), jnp.float32)

  degp = _degree_kernel(ch, full_ch, tail, np_pad)(src_p, dst_p, zeros_deg)

  norms = pl.pallas_call(
      _norm_body,
      out_shape=jax.ShapeDtypeStruct((2, np_pad // K, K), jnp.float32),
  )(degp)

  cp = _coef_kernel(ch, np_pad)(src_p, dst_p, ew_p, norms)


  layer = _layer_kernel(n, np_pad, d, ch)
  bn = 1000 if n % 1000 == 0 else n // NS

  part1 = layer(feat, src_p, dst_p, cp, zeros_acc)

  h1, pool1 = _matmul_call(part1, W0, a0, n, d, bn)
  part2 = layer(h1, src_p, dst_p, cp, zeros_acc)
  h2, pool2 = _matmul_call(part2, W1, a1, n, d, bn)
  hg = jnp.concatenate([pool1, pool2], axis=-1)
  return (h2, hg)
